# baseline (speedup 1.0000x reference)
import functools

import jax
import jax.numpy as jnp
from jax import lax
from jax.experimental import pallas as pl
from jax.experimental.pallas import tpu as pltpu
from jax.experimental.pallas import tpu_sc as plsc

_NC = 2
_NS = 16
_CHUNK = 100


@functools.lru_cache(maxsize=None)
def _make_sc_segment_sums(N, D, E):
    NS, NC, C = _NS, _NC, _CHUNK
    EPW = E // NS
    NBLK = 17
    BCH = 12
    NP = N + 8
    SROW_T = 10
    ROWS_T = N // SROW_T
    WCH = 96
    NW_FULL = ROWS_T // WCH
    W_TAIL = ROWS_T - NW_FULL * WCH
    DEG_T = 2000

    mesh = plsc.VectorSubcoreMesh(core_axis_name="c", subcore_axis_name="s")

    @functools.partial(
        pl.kernel,
        out_type=(
            jax.ShapeDtypeStruct((NC, N, D), jnp.float32),
            jax.ShapeDtypeStruct((N,), jnp.float32),
            jax.ShapeDtypeStruct((N,), jnp.float32),
        ),
        mesh=mesh,
        scratch_types=[
            pltpu.VMEM((BCH, C), jnp.int32),
            pltpu.VMEM((BCH, C), jnp.int32),
            pltpu.VMEM((BCH, C), jnp.int32),
            pltpu.VMEM((BCH, C), jnp.int32),
            pltpu.VMEM((C, D), jnp.float32),
            pltpu.VMEM((C, D), jnp.float32),
            pltpu.VMEM((C, D), jnp.float32),
            pltpu.VMEM((128,), jnp.float32),
            pltpu.VMEM((DEG_T,), jnp.float32),
            pltpu.VMEM_SHARED((NP, D), jnp.float32),
            pltpu.VMEM_SHARED((NP,), jnp.float32),
            [pltpu.SemaphoreType.DMA] * 3,
            [pltpu.SemaphoreType.DMA] * 3,
            [pltpu.SemaphoreType.DMA] * 3,
            pltpu.SemaphoreType.DMA,
        ],
    )
    def sc_kernel(nf_hbm, edges_hbm, s_out, deg_f_out, deg_r_out,
                  gidx_a, gidx_b, sidx_a, sidx_b, rows_0, rows_1, rows_2,
                  ones_v, dstage, s_acc, deg_acc, gsems, ssems, dsems, isem):
        rows = (rows_0, rows_1, rows_2)
        c = lax.axis_index("c")
        s = lax.axis_index("s")

        zero16 = jnp.zeros((16,), jnp.float32)
        one16 = jnp.ones((16,), jnp.float32)
        for j in range(128 // 16):
            ones_v[pl.ds(j * 16, 16)] = one16

        def _zrow(i, carry):
            for j in range(D // 16):
                rows_0[i, pl.ds(j * 16, 16)] = zero16
            return carry

        lax.fori_loop(0, C, _zrow, 0)

        def _zdeg(i, carry):
            dstage[pl.ds(i * 16, 16)] = zero16
            return carry

        lax.fori_loop(0, DEG_T // 16, _zdeg, 0)

        @pl.when(s < SROW_T)
        def _():
            for k in range(NW_FULL):
                pltpu.sync_copy(
                    rows_0.at[pl.ds(0, WCH)],
                    s_acc.at[pl.ds(s * ROWS_T + k * WCH, WCH)])
            if W_TAIL:
                pltpu.sync_copy(
                    rows_0.at[pl.ds(0, W_TAIL)],
                    s_acc.at[pl.ds(s * ROWS_T + NW_FULL * WCH, W_TAIL)])

        @pl.when(s < N // DEG_T)
        def _():
            pltpu.sync_copy(dstage, deg_acc.at[pl.ds(s * DEG_T, DEG_T)])

        plsc.subcore_barrier()

        g = c
        r = 1 - c

        def _gather(idx, l, b):
            pltpu.async_copy(nf_hbm.at[idx.at[l]], rows[b], gsems[b])

        def _gwait(b):
            pltpu.make_async_copy(nf_hbm.at[gidx_a.at[0]], rows[b],
                                  gsems[b]).wait()

        def _scat(idx, l, b):
            pltpu.async_copy(rows[b], s_acc.at[idx.at[l]], ssems[b], add=True)
            pltpu.async_copy(
                ones_v.at[pl.ds(0, C)], deg_acc.at[idx.at[l]], dsems[b],
                add=True)

        def _swait(b):
            pltpu.make_async_copy(rows[b], s_acc.at[sidx_a.at[0]],
                                  ssems[b]).wait()
            pltpu.make_async_copy(
                ones_v.at[pl.ds(0, C)], deg_acc.at[sidx_a.at[0]],
                dsems[b]).wait()

        def _run_block(gc, sc_, gn, sn, blk, first):
            for l in range(BCH):
                b = l % 3
                p = (b + 2) % 3
                if not (first and l == 0):
                    _swait(p)
                if l < BCH - 2:
                    _gather(gc, l + 2, p)
                else:
                    _gather(gn, l - (BCH - 2), p)
                if l == 0:
                    nb = jnp.minimum(blk + 1, NBLK - 1)
                    pltpu.async_copy(edges_hbm.at[g, s, nb], gn, isem)
                    pltpu.async_copy(edges_hbm.at[r, s, nb], sn, isem)
                if l == BCH - 3:
                    pltpu.make_async_copy(edges_hbm.at[g, s, 0], gn,
                                          isem).wait()
                    pltpu.make_async_copy(edges_hbm.at[r, s, 0], sn,
                                          isem).wait()
                _gwait(b)
                _scat(sc_, l, b)

        pltpu.sync_copy(edges_hbm.at[g, s, 0], gidx_a)
        pltpu.sync_copy(edges_hbm.at[r, s, 0], sidx_a)
        _gather(gidx_a, 0, 0)
        _gather(gidx_a, 1, 1)
        _run_block(gidx_a, sidx_a, gidx_b, sidx_b, 0, True)

        def _pair(pp, carry):
            b0 = 2 * pp + 1
            _run_block(gidx_b, sidx_b, gidx_a, sidx_a, b0, False)
            _run_block(gidx_a, sidx_a, gidx_b, sidx_b, b0 + 1, False)
            return carry

        lax.fori_loop(0, (NBLK - 1) // 2, _pair, 0)

        _swait(2)
        _gwait(0)
        _gwait(1)

        plsc.subcore_barrier()

        @pl.when(s < SROW_T)
        def _():
            for k in range(NW_FULL):
                lo = s * ROWS_T + k * WCH
                pltpu.sync_copy(s_acc.at[pl.ds(lo, WCH)], rows_0.at[pl.ds(0, WCH)])
                pltpu.sync_copy(rows_0.at[pl.ds(0, WCH)], s_out.at[c, pl.ds(lo, WCH)])
            if W_TAIL:
                lo = s * ROWS_T + NW_FULL * WCH
                pltpu.sync_copy(
                    s_acc.at[pl.ds(lo, W_TAIL)], rows_1.at[pl.ds(0, W_TAIL)])
                pltpu.sync_copy(
                    rows_1.at[pl.ds(0, W_TAIL)], s_out.at[c, pl.ds(lo, W_TAIL)])

        @pl.when(s < N // DEG_T)
        def _():
            pltpu.sync_copy(deg_acc.at[pl.ds(s * DEG_T, DEG_T)], dstage)

            @pl.when(c == 0)
            def _():
                pltpu.sync_copy(dstage, deg_f_out.at[pl.ds(s * DEG_T, DEG_T)])

            @pl.when(c == 1)
            def _():
                pltpu.sync_copy(dstage, deg_r_out.at[pl.ds(s * DEG_T, DEG_T)])

    return sc_kernel


@functools.lru_cache(maxsize=None)
def _make_tc_combine(N, D, RPAD):
    R = 400
    G = N // R
    dn = (((1,), (1,)), ((), ()))
    f32 = jnp.float32

    def body(nf, sf, sr, df, dr, rp, wo, wi, ws, wr, bo, bi, bs, br,
             out, rout):
        i = pl.program_id(0)
        rp_v = rp[...]
        rw_o = lax.dot_general(rp_v, wo[...], dn, preferred_element_type=f32)
        rw_i = lax.dot_general(rp_v, wi[...], dn, preferred_element_type=f32)
        rw_s = lax.dot_general(rp_v, ws[...], dn, preferred_element_type=f32)
        c_f = bo[...] - rw_o[1:2, :]
        c_r = bi[...] - rw_i[2:3, :]
        c_s = bs[...] - rw_s[2:3, :]
        df_v = df[...]
        dr_v = dr[...]
        a_f = sf[...] * (1.0 / jnp.maximum(df_v, 1.0))
        a_r = sr[...] * (1.0 / jnp.maximum(dr_v, 1.0))
        acc = lax.dot_general(a_f, wo[...], dn, preferred_element_type=f32)
        acc += lax.dot_general(a_r, wi[...], dn, preferred_element_type=f32)
        acc += lax.dot_general(nf[...], ws[...], dn, preferred_element_type=f32)
        ind_f = jnp.where(df_v > 0.0, 1.0, 0.0)
        ind_r = jnp.where(dr_v > 0.0, 1.0, 0.0)
        out[...] = acc + ind_f * c_f + ind_r * c_r + c_s

        @pl.when(i == 0)
        def _():
            rout[...] = (
                lax.dot_general(rp_v, wr[...], dn, preferred_element_type=f32)
                + br[...]
            )

    row_blk = pl.BlockSpec((R, D), lambda i: (i, 0))
    col_blk = pl.BlockSpec((R, 1), lambda i: (i, 0))
    full = lambda shape: pl.BlockSpec(shape, lambda i: (0,) * len(shape))

    return pl.pallas_call(
        body,
        grid=(G,),
        in_specs=[
            row_blk, row_blk, row_blk, col_blk, col_blk,
            full((RPAD, D)),
            full((D, D)), full((D, D)), full((D, D)), full((D, D)),
            full((1, D)), full((1, D)), full((1, D)), full((1, D)),
        ],
        out_specs=[row_blk, full((RPAD, D))],
        out_shape=(
            jax.ShapeDtypeStruct((N, D), f32),
            jax.ShapeDtypeStruct((RPAD, D), f32),
        ),
    )


def kernel(n_feats, r_feats, edge_index, W_O_w, W_O_b, W_I_w, W_I_b,
           W_S_w, W_S_b, W_R_w, W_R_b):
    N, D = n_feats.shape
    E = edge_index.shape[1]
    NR = r_feats.shape[0]
    RPAD = 8

    EPW = E // _NS
    NBLK, BCH = 17, 12
    epw_pad = NBLK * BCH * _CHUNK
    ei = edge_index.reshape(2, _NS, EPW)
    pad = N + (jnp.arange(epw_pad - EPW, dtype=jnp.int32) % 8)
    pad = jnp.broadcast_to(pad, (2, _NS, epw_pad - EPW))
    edges_r = jnp.concatenate([ei, pad], axis=2).reshape(
        2, _NS, NBLK, BCH, _CHUNK)
    nf_p = jnp.concatenate(
        [n_feats, jnp.zeros((8, D), n_feats.dtype)], axis=0)
    S, deg_f, deg_r = _make_sc_segment_sums(N, D, E)(nf_p, edges_r)

    rp = jnp.zeros((RPAD, D), jnp.float32).at[:NR].set(r_feats)
    n_out, r_out = _make_tc_combine(N, D, RPAD)(
        n_feats,
        S[0], S[1],
        deg_f.reshape(N, 1), deg_r.reshape(N, 1),
        rp,
        W_O_w, W_I_w, W_S_w, W_R_w,
        W_O_b.reshape(1, D), W_I_b.reshape(1, D),
        W_S_b.reshape(1, D), W_R_b.reshape(1, D),
    )
    return n_out, r_out[:NR]


# speedup vs baseline: 1.1854x; 1.1854x over previous
import functools

import jax
import jax.numpy as jnp
from jax import lax
from jax.experimental import pallas as pl
from jax.experimental.pallas import tpu as pltpu
from jax.experimental.pallas import tpu_sc as plsc

_NC = 2
_NS = 16
_CHUNK = 100


@functools.lru_cache(maxsize=None)
def _make_sc_segment_sums(N, D, E):
    NS, NC, C = _NS, _NC, _CHUNK
    EPW = E // NS
    NCH = EPW // C
    NBLK = 8
    BCH = NCH // NBLK
    SROW_T = 10
    ROWS_T = N // SROW_T
    WCH = 96
    NW_FULL = ROWS_T // WCH
    W_TAIL = ROWS_T - NW_FULL * WCH
    DEG_T = 2000

    mesh = plsc.VectorSubcoreMesh(core_axis_name="c", subcore_axis_name="s")

    @functools.partial(
        pl.kernel,
        out_type=(
            jax.ShapeDtypeStruct((NC, N, D), jnp.float32),
            jax.ShapeDtypeStruct((N,), jnp.float32),
            jax.ShapeDtypeStruct((N,), jnp.float32),
        ),
        mesh=mesh,
        scratch_types=[
            pltpu.VMEM((BCH, C), jnp.int32),
            pltpu.VMEM((BCH, C), jnp.int32),
            pltpu.VMEM((C, D), jnp.float32),
            pltpu.VMEM((C, D), jnp.float32),
            pltpu.VMEM((C, D), jnp.float32),
            pltpu.VMEM((128,), jnp.float32),
            pltpu.VMEM((DEG_T,), jnp.float32),
            pltpu.VMEM_SHARED((N, D), jnp.float32),
            pltpu.VMEM_SHARED((N,), jnp.float32),
            [pltpu.SemaphoreType.DMA] * 3,
            [pltpu.SemaphoreType.DMA] * 3,
            [pltpu.SemaphoreType.DMA] * 3,
        ],
    )
    def sc_kernel(nf_hbm, edges_hbm, s_out, deg_f_out, deg_r_out,
                  gidx, sidx, rows_0, rows_1, rows_2, ones_v, dstage,
                  s_acc, deg_acc, gsems, ssems, dsems):
        rows = (rows_0, rows_1, rows_2)
        c = lax.axis_index("c")
        s = lax.axis_index("s")

        zero16 = jnp.zeros((16,), jnp.float32)
        one16 = jnp.ones((16,), jnp.float32)
        for j in range(128 // 16):
            ones_v[pl.ds(j * 16, 16)] = one16

        def _zrow(i, carry):
            for j in range(D // 16):
                rows_0[i, pl.ds(j * 16, 16)] = zero16
            return carry

        lax.fori_loop(0, C, _zrow, 0)

        def _zdeg(i, carry):
            dstage[pl.ds(i * 16, 16)] = zero16
            return carry

        lax.fori_loop(0, DEG_T // 16, _zdeg, 0)

        @pl.when(s < SROW_T)
        def _():
            for k in range(NW_FULL):
                pltpu.sync_copy(
                    rows_0.at[pl.ds(0, WCH)],
                    s_acc.at[pl.ds(s * ROWS_T + k * WCH, WCH)])
            if W_TAIL:
                pltpu.sync_copy(
                    rows_0.at[pl.ds(0, W_TAIL)],
                    s_acc.at[pl.ds(s * ROWS_T + NW_FULL * WCH, W_TAIL)])

        @pl.when(s < N // DEG_T)
        def _():
            pltpu.sync_copy(dstage, deg_acc.at[pl.ds(s * DEG_T, DEG_T)])

        plsc.subcore_barrier()

        g = c
        r = 1 - c

        def _gather(j, b, buf):
            pltpu.async_copy(nf_hbm.at[gidx.at[j]], buf, gsems[b])

        def _gwait(j, b, buf):
            pltpu.make_async_copy(nf_hbm.at[gidx.at[j]], buf, gsems[b]).wait()

        def _scat(j, b, buf):
            pltpu.async_copy(buf, s_acc.at[sidx.at[j]], ssems[b], add=True)
            pltpu.async_copy(
                ones_v.at[pl.ds(0, C)], deg_acc.at[sidx.at[j]], dsems[b],
                add=True)

        def _swait(j, b, buf):
            pltpu.make_async_copy(buf, s_acc.at[sidx.at[j]], ssems[b]).wait()
            pltpu.make_async_copy(
                ones_v.at[pl.ds(0, C)], deg_acc.at[sidx.at[j]], dsems[b]).wait()

        def _step(j, b, issue_next):
            if issue_next:
                p = (b + 2) % 3
                _swait(j - 1, p, rows[p])
                _gather(j + 2, p, rows[p])
            _gwait(j, b, rows[b])
            _scat(j, b, rows[b])

        def _block(blk, carry):
            pltpu.sync_copy(edges_hbm.at[g, s, blk], gidx)
            pltpu.sync_copy(edges_hbm.at[r, s, blk], sidx)

            _gather(0, 0, rows_0)
            _gather(1, 1, rows_1)
            _gather(2, 2, rows_2)
            _gwait(0, 0, rows_0)
            _scat(0, 0, rows_0)
            _step(jnp.int32(1), 1, True)

            def _body(jj, carry2):
                j = 3 * jj + 2
                _step(j, 2, True)
                _step(j + 1, 0, True)
                _step(j + 2, 1, True)
                return carry2

            lax.fori_loop(0, (BCH - 4) // 3, _body, 0)

            _step(jnp.int32(BCH - 2), (BCH - 2) % 3, False)
            _step(jnp.int32(BCH - 1), (BCH - 1) % 3, False)
            for j in (BCH - 3, BCH - 2, BCH - 1):
                _swait(j, j % 3, rows[j % 3])
            return carry

        lax.fori_loop(0, NBLK, _block, 0)

        plsc.subcore_barrier()

        @pl.when(s < SROW_T)
        def _():
            for k in range(NW_FULL):
                lo = s * ROWS_T + k * WCH
                pltpu.sync_copy(s_acc.at[pl.ds(lo, WCH)], rows_0.at[pl.ds(0, WCH)])
                pltpu.sync_copy(rows_0.at[pl.ds(0, WCH)], s_out.at[c, pl.ds(lo, WCH)])
            if W_TAIL:
                lo = s * ROWS_T + NW_FULL * WCH
                pltpu.sync_copy(
                    s_acc.at[pl.ds(lo, W_TAIL)], rows_1.at[pl.ds(0, W_TAIL)])
                pltpu.sync_copy(
                    rows_1.at[pl.ds(0, W_TAIL)], s_out.at[c, pl.ds(lo, W_TAIL)])

        @pl.when(s < N // DEG_T)
        def _():
            pltpu.sync_copy(deg_acc.at[pl.ds(s * DEG_T, DEG_T)], dstage)

            @pl.when(c == 0)
            def _():
                pltpu.sync_copy(dstage, deg_f_out.at[pl.ds(s * DEG_T, DEG_T)])

            @pl.when(c == 1)
            def _():
                pltpu.sync_copy(dstage, deg_r_out.at[pl.ds(s * DEG_T, DEG_T)])

    return sc_kernel


@functools.lru_cache(maxsize=None)
def _make_tc_combine(N, D, RPAD):
    R = 400
    G = N // R
    dn = (((1,), (1,)), ((), ()))
    f32 = jnp.float32

    def body(nf, sf, sr, df, dr, rp, wo, wi, ws, wr, bo, bi, bs, br,
             out, rout):
        i = pl.program_id(0)
        rp_v = rp[...]
        rw_o = lax.dot_general(rp_v, wo[...], dn, preferred_element_type=f32)
        rw_i = lax.dot_general(rp_v, wi[...], dn, preferred_element_type=f32)
        rw_s = lax.dot_general(rp_v, ws[...], dn, preferred_element_type=f32)
        c_f = bo[...] - rw_o[1:2, :]
        c_r = bi[...] - rw_i[2:3, :]
        c_s = bs[...] - rw_s[2:3, :]
        df_v = df[...]
        dr_v = dr[...]
        a_f = sf[...] * (1.0 / jnp.maximum(df_v, 1.0))
        a_r = sr[...] * (1.0 / jnp.maximum(dr_v, 1.0))
        acc = lax.dot_general(a_f, wo[...], dn, preferred_element_type=f32)
        acc += lax.dot_general(a_r, wi[...], dn, preferred_element_type=f32)
        acc += lax.dot_general(nf[...], ws[...], dn, preferred_element_type=f32)
        ind_f = jnp.where(df_v > 0.0, 1.0, 0.0)
        ind_r = jnp.where(dr_v > 0.0, 1.0, 0.0)
        out[...] = acc + ind_f * c_f + ind_r * c_r + c_s

        @pl.when(i == 0)
        def _():
            rout[...] = (
                lax.dot_general(rp_v, wr[...], dn, preferred_element_type=f32)
                + br[...]
            )

    row_blk = pl.BlockSpec((R, D), lambda i: (i, 0))
    col_blk = pl.BlockSpec((R, 1), lambda i: (i, 0))
    full = lambda shape: pl.BlockSpec(shape, lambda i: (0,) * len(shape))

    return pl.pallas_call(
        body,
        grid=(G,),
        in_specs=[
            row_blk, row_blk, row_blk, col_blk, col_blk,
            full((RPAD, D)),
            full((D, D)), full((D, D)), full((D, D)), full((D, D)),
            full((1, D)), full((1, D)), full((1, D)), full((1, D)),
        ],
        out_specs=[row_blk, full((RPAD, D))],
        out_shape=(
            jax.ShapeDtypeStruct((N, D), f32),
            jax.ShapeDtypeStruct((RPAD, D), f32),
        ),
    )


def kernel(n_feats, r_feats, edge_index, W_O_w, W_O_b, W_I_w, W_I_b,
           W_S_w, W_S_b, W_R_w, W_R_b):
    N, D = n_feats.shape
    E = edge_index.shape[1]
    NR = r_feats.shape[0]
    RPAD = 8

    NCH = (E // _NS) // _CHUNK
    edges_r = edge_index.reshape(2, _NS, 8, NCH // 8, _CHUNK)
    S, deg_f, deg_r = _make_sc_segment_sums(N, D, E)(n_feats, edges_r)

    rp = jnp.zeros((RPAD, D), jnp.float32).at[:NR].set(r_feats)
    n_out, r_out = _make_tc_combine(N, D, RPAD)(
        n_feats,
        S[0], S[1],
        deg_f.reshape(N, 1), deg_r.reshape(N, 1),
        rp,
        W_O_w, W_I_w, W_S_w, W_R_w,
        W_O_b.reshape(1, D), W_I_b.reshape(1, D),
        W_S_b.reshape(1, D), W_R_b.reshape(1, D),
    )
    return n_out, r_out[:NR]


# speedup vs baseline: 1.2114x; 1.0220x over previous
import functools

import jax
import jax.numpy as jnp
from jax import lax
from jax.experimental import pallas as pl
from jax.experimental.pallas import tpu as pltpu
from jax.experimental.pallas import tpu_sc as plsc

_NC = 2
_NS = 16
_CHUNK = 100


@functools.lru_cache(maxsize=None)
def _make_sc_segment_sums(N, D, E):
    NS, NC, C = _NS, _NC, _CHUNK
    EPW = E // NS
    NCH = EPW // C
    NBLK = 8
    BCH = NCH // NBLK
    SROW_T = 10
    ROWS_T = N // SROW_T
    WCH = 96
    NW_FULL = ROWS_T // WCH
    W_TAIL = ROWS_T - NW_FULL * WCH
    DEG_T = 2000

    mesh = plsc.VectorSubcoreMesh(core_axis_name="c", subcore_axis_name="s")

    @functools.partial(
        pl.kernel,
        out_type=(
            jax.ShapeDtypeStruct((NC, N, D), jnp.float32),
            jax.ShapeDtypeStruct((N,), jnp.float32),
            jax.ShapeDtypeStruct((N,), jnp.float32),
        ),
        mesh=mesh,
        scratch_types=[
            pltpu.VMEM((BCH, C), jnp.int32),
            pltpu.VMEM((BCH, C), jnp.int32),
            pltpu.VMEM((C, D), jnp.float32),
            pltpu.VMEM((C, D), jnp.float32),
            pltpu.VMEM((C, D), jnp.float32),
            pltpu.VMEM((128,), jnp.float32),
            pltpu.VMEM((DEG_T,), jnp.float32),
            pltpu.VMEM_SHARED((N, D), jnp.float32),
            pltpu.VMEM_SHARED((N,), jnp.float32),
            [pltpu.SemaphoreType.DMA] * 3,
            [pltpu.SemaphoreType.DMA] * 3,
            [pltpu.SemaphoreType.DMA] * 3,
        ],
    )
    def sc_kernel(nf_hbm, edges_hbm, s_out, deg_f_out, deg_r_out,
                  gidx, sidx, rows_0, rows_1, rows_2, ones_v, dstage,
                  s_acc, deg_acc, gsems, ssems, dsems):
        rows = (rows_0, rows_1, rows_2)
        c = lax.axis_index("c")
        s = lax.axis_index("s")

        zero16 = jnp.zeros((16,), jnp.float32)
        one16 = jnp.ones((16,), jnp.float32)
        for j in range(128 // 16):
            ones_v[pl.ds(j * 16, 16)] = one16

        def _zrow(i, carry):
            for j in range(D // 16):
                rows_0[i, pl.ds(j * 16, 16)] = zero16
            return carry

        lax.fori_loop(0, C, _zrow, 0)

        def _zdeg(i, carry):
            dstage[pl.ds(i * 16, 16)] = zero16
            return carry

        lax.fori_loop(0, DEG_T // 16, _zdeg, 0)

        @pl.when(s < SROW_T)
        def _():
            for k in range(NW_FULL):
                pltpu.async_copy(
                    rows_0.at[pl.ds(0, WCH)],
                    s_acc.at[pl.ds(s * ROWS_T + k * WCH, WCH)], ssems[0])
            if W_TAIL:
                pltpu.async_copy(
                    rows_0.at[pl.ds(0, W_TAIL)],
                    s_acc.at[pl.ds(s * ROWS_T + NW_FULL * WCH, W_TAIL)],
                    ssems[0])
            for k in range(NW_FULL):
                pltpu.make_async_copy(
                    rows_0.at[pl.ds(0, WCH)], s_acc.at[pl.ds(0, WCH)],
                    ssems[0]).wait()
            if W_TAIL:
                pltpu.make_async_copy(
                    rows_0.at[pl.ds(0, W_TAIL)], s_acc.at[pl.ds(0, W_TAIL)],
                    ssems[0]).wait()

        @pl.when(s < N // DEG_T)
        def _():
            pltpu.sync_copy(dstage, deg_acc.at[pl.ds(s * DEG_T, DEG_T)])

        plsc.subcore_barrier()

        g = c
        r = 1 - c

        def _gather(j, b, buf):
            pltpu.async_copy(nf_hbm.at[gidx.at[j]], buf, gsems[b])

        def _gwait(j, b, buf):
            pltpu.make_async_copy(nf_hbm.at[gidx.at[j]], buf, gsems[b]).wait()

        def _scat(j, b, buf):
            pltpu.async_copy(buf, s_acc.at[sidx.at[j]], ssems[b], add=True)
            pltpu.async_copy(
                ones_v.at[pl.ds(0, C)], deg_acc.at[sidx.at[j]], dsems[b],
                add=True)

        def _swait(j, b, buf):
            pltpu.make_async_copy(buf, s_acc.at[sidx.at[j]], ssems[b]).wait()
            pltpu.make_async_copy(
                ones_v.at[pl.ds(0, C)], deg_acc.at[sidx.at[j]], dsems[b]).wait()

        def _step(j, b, issue_next):
            if issue_next:
                p = (b + 2) % 3
                _swait(j - 1, p, rows[p])
                _gather(j + 2, p, rows[p])
            _gwait(j, b, rows[b])
            _scat(j, b, rows[b])

        def _block(blk, carry):
            pltpu.sync_copy(edges_hbm.at[g, s, blk], gidx)
            pltpu.sync_copy(edges_hbm.at[r, s, blk], sidx)

            _gather(0, 0, rows_0)
            _gather(1, 1, rows_1)
            _gather(2, 2, rows_2)
            _gwait(0, 0, rows_0)
            _scat(0, 0, rows_0)
            _step(jnp.int32(1), 1, True)

            def _body(jj, carry2):
                j = 3 * jj + 2
                _step(j, 2, True)
                _step(j + 1, 0, True)
                _step(j + 2, 1, True)
                return carry2

            lax.fori_loop(0, (BCH - 4) // 3, _body, 0)

            _step(jnp.int32(BCH - 2), (BCH - 2) % 3, False)
            _step(jnp.int32(BCH - 1), (BCH - 1) % 3, False)
            for j in (BCH - 3, BCH - 2, BCH - 1):
                _swait(j, j % 3, rows[j % 3])
            return carry

        lax.fori_loop(0, NBLK, _block, 0)

        plsc.subcore_barrier()

        @pl.when(s < SROW_T)
        def _():
            bufs = (rows_0, rows_1)
            for k in range(NW_FULL):
                buf = bufs[k % 2]
                lo = s * ROWS_T + k * WCH
                if k >= 2:
                    pltpu.make_async_copy(
                        buf.at[pl.ds(0, WCH)], s_out.at[0, pl.ds(0, WCH)],
                        ssems[k % 2]).wait()
                pltpu.sync_copy(s_acc.at[pl.ds(lo, WCH)], buf.at[pl.ds(0, WCH)])
                pltpu.async_copy(
                    buf.at[pl.ds(0, WCH)], s_out.at[c, pl.ds(lo, WCH)],
                    ssems[k % 2])
            if W_TAIL:
                lo = s * ROWS_T + NW_FULL * WCH
                pltpu.sync_copy(
                    s_acc.at[pl.ds(lo, W_TAIL)], rows_2.at[pl.ds(0, W_TAIL)])
                pltpu.async_copy(
                    rows_2.at[pl.ds(0, W_TAIL)], s_out.at[c, pl.ds(lo, W_TAIL)],
                    ssems[2])
            for b in (0, 1):
                pltpu.make_async_copy(
                    bufs[b].at[pl.ds(0, WCH)], s_out.at[0, pl.ds(0, WCH)],
                    ssems[b]).wait()
            if W_TAIL:
                pltpu.make_async_copy(
                    rows_2.at[pl.ds(0, W_TAIL)],
                    s_out.at[0, pl.ds(0, W_TAIL)], ssems[2]).wait()

        @pl.when(s < N // DEG_T)
        def _():
            pltpu.sync_copy(deg_acc.at[pl.ds(s * DEG_T, DEG_T)], dstage)

            @pl.when(c == 0)
            def _():
                pltpu.sync_copy(dstage, deg_f_out.at[pl.ds(s * DEG_T, DEG_T)])

            @pl.when(c == 1)
            def _():
                pltpu.sync_copy(dstage, deg_r_out.at[pl.ds(s * DEG_T, DEG_T)])

    return sc_kernel


@functools.lru_cache(maxsize=None)
def _make_tc_combine(N, D, RPAD):
    R = 400
    G = N // R
    dn = (((1,), (1,)), ((), ()))
    f32 = jnp.float32

    def body(nf, sf, sr, df, dr, rp, wo, wi, ws, wr, bo, bi, bs, br,
             out, rout):
        i = pl.program_id(0)
        rp_v = rp[...]
        rw_o = lax.dot_general(rp_v, wo[...], dn, preferred_element_type=f32)
        rw_i = lax.dot_general(rp_v, wi[...], dn, preferred_element_type=f32)
        rw_s = lax.dot_general(rp_v, ws[...], dn, preferred_element_type=f32)
        c_f = bo[...] - rw_o[1:2, :]
        c_r = bi[...] - rw_i[2:3, :]
        c_s = bs[...] - rw_s[2:3, :]
        df_v = df[...]
        dr_v = dr[...]
        a_f = sf[...] * (1.0 / jnp.maximum(df_v, 1.0))
        a_r = sr[...] * (1.0 / jnp.maximum(dr_v, 1.0))
        acc = lax.dot_general(a_f, wo[...], dn, preferred_element_type=f32)
        acc += lax.dot_general(a_r, wi[...], dn, preferred_element_type=f32)
        acc += lax.dot_general(nf[...], ws[...], dn, preferred_element_type=f32)
        ind_f = jnp.where(df_v > 0.0, 1.0, 0.0)
        ind_r = jnp.where(dr_v > 0.0, 1.0, 0.0)
        out[...] = acc + ind_f * c_f + ind_r * c_r + c_s

        @pl.when(i == 0)
        def _():
            rout[...] = (
                lax.dot_general(rp_v, wr[...], dn, preferred_element_type=f32)
                + br[...]
            )

    row_blk = pl.BlockSpec((R, D), lambda i: (i, 0))
    col_blk = pl.BlockSpec((R, 1), lambda i: (i, 0))
    full = lambda shape: pl.BlockSpec(shape, lambda i: (0,) * len(shape))

    return pl.pallas_call(
        body,
        grid=(G,),
        in_specs=[
            row_blk, row_blk, row_blk, col_blk, col_blk,
            full((RPAD, D)),
            full((D, D)), full((D, D)), full((D, D)), full((D, D)),
            full((1, D)), full((1, D)), full((1, D)), full((1, D)),
        ],
        out_specs=[row_blk, full((RPAD, D))],
        out_shape=(
            jax.ShapeDtypeStruct((N, D), f32),
            jax.ShapeDtypeStruct((RPAD, D), f32),
        ),
    )


def kernel(n_feats, r_feats, edge_index, W_O_w, W_O_b, W_I_w, W_I_b,
           W_S_w, W_S_b, W_R_w, W_R_b):
    N, D = n_feats.shape
    E = edge_index.shape[1]
    NR = r_feats.shape[0]
    RPAD = 8

    NCH = (E // _NS) // _CHUNK
    edges_r = edge_index.reshape(2, _NS, 8, NCH // 8, _CHUNK)
    S, deg_f, deg_r = _make_sc_segment_sums(N, D, E)(n_feats, edges_r)

    rp = jnp.zeros((RPAD, D), jnp.float32).at[:NR].set(r_feats)
    n_out, r_out = _make_tc_combine(N, D, RPAD)(
        n_feats,
        S[0], S[1],
        deg_f.reshape(N, 1), deg_r.reshape(N, 1),
        rp,
        W_O_w, W_I_w, W_S_w, W_R_w,
        W_O_b.reshape(1, D), W_I_b.reshape(1, D),
        W_S_b.reshape(1, D), W_R_b.reshape(1, D),
    )
    return n_out, r_out[:NR]
